# baseline (device time: 113067 ns/iter reference)
import numpy as np
import jax
import jax.numpy as jnp
from jax import lax
from jax.experimental import pallas as pl
from jax.experimental.pallas import tpu as pltpu

N_DEV = 4
B = 2
S_LOC = 512
D = 1024
H = 8
DH = 128
ROWS = B * S_LOC
SCALE = 0.08838834764831843

F32 = jnp.float32
BF16 = jnp.bfloat16
FP8 = jnp.float8_e4m3fn

OWN, FROM_L, FROM_R, DIAG = 0, 1, 2, 3


def _rot_mat() -> np.ndarray:
    P = np.zeros((DH, DH), np.float32)
    for i in range(DH // 2):
        P[2 * i + 1, 2 * i] = -1.0
        P[2 * i, 2 * i + 1] = 1.0
    return P


def kernel(x, Wq, Wk, Wv, Wo):
    my = lax.axis_index("i")

    inv = (1.0 / (10000.0 ** (np.arange(0, DH, 2) / DH))).astype(np.float32)
    pos = (my * S_LOC + jnp.arange(S_LOC)).astype(F32)
    ang = pos[:, None] * inv[None, :]
    cos = jnp.repeat(jnp.cos(ang), 2, axis=-1)
    sin = jnp.repeat(jnp.sin(ang), 2, axis=-1)
    P = jnp.asarray(_rot_mat())

    def body(x_ref, wq_hbm, wk_hbm, wv_hbm, wo_hbm, cos_ref, sin_ref, p_ref,
             out_ref, comm_ref, q_ref, w_ref, den_ref, p_buf,
             send_sems, recv_sems, dma_sem):
        my_pos = lax.axis_index("i")
        right = (my_pos + 1) % N_DEV
        left = (my_pos - 1) % N_DEV

        barrier = pltpu.get_barrier_semaphore()
        for nbr in (left, right):
            pl.semaphore_signal(barrier, inc=1, device_id=(nbr,),
                                device_id_type=pl.DeviceIdType.MESH)
        pl.semaphore_wait(barrier, 2)

        def load_w(src_hbm):
            cp = pltpu.make_async_copy(src_hbm, w_ref, dma_sem)
            cp.start()
            cp.wait()

        x2 = x_ref[...].reshape(ROWS, D).astype(BF16)
        cs = jnp.concatenate([cos_ref[...], cos_ref[...]], axis=0)
        sn = jnp.concatenate([sin_ref[...], sin_ref[...]], axis=0)

        def proj():
            return jnp.dot(x2, w_ref[...].astype(BF16),
                           preferred_element_type=F32)

        def rope(t, to):
            outs = []
            for h in range(H):
                th = t[:, h * DH:(h + 1) * DH]
                tr = jnp.dot(th, p_ref[...], preferred_element_type=F32)
                outs.append(th * cs + tr * sn)
            return jnp.concatenate(outs, axis=1).astype(to)

        def remote(src_slot, dst_slot, sem, dev):
            return pltpu.make_async_remote_copy(
                src_ref=comm_ref.at[src_slot[0], src_slot[1]],
                dst_ref=comm_ref.at[dst_slot[0], dst_slot[1]],
                send_sem=send_sems.at[sem], recv_sem=recv_sems.at[sem],
                device_id=(dev,), device_id_type=pl.DeviceIdType.MESH)

        load_w(wk_hbm)
        comm_ref[OWN, 0] = rope(proj(), BF16)
        s_kr = remote((OWN, 0), (FROM_L, 0), 0, right)
        s_kl = remote((OWN, 0), (FROM_R, 0), 1, left)
        s_kr.start()
        s_kl.start()

        load_w(wv_hbm)
        comm_ref[OWN, 1] = proj().astype(BF16)
        s_vr = remote((OWN, 1), (FROM_L, 1), 2, right)
        s_vl = remote((OWN, 1), (FROM_R, 1), 3, left)
        s_vr.start()
        s_vl.start()

        load_w(wq_hbm)
        q_ref[...] = rope(proj(), BF16)
        wo_cp = pltpu.make_async_copy(wo_hbm, w_ref, dma_sem)
        wo_cp.start()

        def flash_tile(slot, b, h):
            r0 = b * S_LOC
            c0 = h * DH
            qbh = q_ref[r0:r0 + S_LOC, c0:c0 + DH]
            k_c = comm_ref[slot, 0, r0:r0 + S_LOC, c0:c0 + DH]
            v_c = comm_ref[slot, 1, r0:r0 + S_LOC, c0:c0 + DH]
            s_c = lax.dot_general(
                qbh, k_c, (((1,), (1,)), ((), ())),
                preferred_element_type=F32) * SCALE
            p_c = jnp.exp(s_c)
            pv = jnp.dot(p_c.astype(BF16), v_c,
                         preferred_element_type=F32)
            ps = jnp.sum(p_c, axis=-1, keepdims=True)
            return pv, ps

        def flash_chunk(slot, first):
            for b in range(B):
                r0 = b * S_LOC
                for h in range(H):
                    c0 = h * DH
                    pv, ps = flash_tile(slot, b, h)
                    if first:
                        out_ref[b, :, c0:c0 + DH] = pv
                        den_ref[r0:r0 + S_LOC, h:h + 1] = ps
                    else:
                        out_ref[b, :, c0:c0 + DH] = (
                            out_ref[b, :, c0:c0 + DH] + pv)
                        den_ref[r0:r0 + S_LOC, h:h + 1] = (
                            den_ref[r0:r0 + S_LOC, h:h + 1] + ps)

        flash_chunk(OWN, first=True)

        def score_chunk(slot, pslot):
            for b in range(B):
                r0 = b * S_LOC
                for h in range(H):
                    c0 = h * DH
                    qbh = q_ref[r0:r0 + S_LOC, c0:c0 + DH]
                    k_c = comm_ref[slot, 0, r0:r0 + S_LOC, c0:c0 + DH]
                    s_c = lax.dot_general(
                        qbh, k_c, (((1,), (1,)), ((), ())),
                        preferred_element_type=F32) * SCALE
                    p_c = jnp.exp(s_c)
                    p_buf[pslot, r0:r0 + S_LOC,
                          h * S_LOC:(h + 1) * S_LOC] = p_c.astype(BF16)
                    den_ref[r0:r0 + S_LOC, h:h + 1] = (
                        den_ref[r0:r0 + S_LOC, h:h + 1]
                        + jnp.sum(p_c, axis=-1, keepdims=True))

        def pv_chunk(slot, pslot):
            for b in range(B):
                r0 = b * S_LOC
                for h in range(H):
                    c0 = h * DH
                    v_c = comm_ref[slot, 1, r0:r0 + S_LOC, c0:c0 + DH]
                    p_c = p_buf[pslot, r0:r0 + S_LOC,
                                h * S_LOC:(h + 1) * S_LOC]
                    out_ref[b, :, c0:c0 + DH] = (
                        out_ref[b, :, c0:c0 + DH]
                        + jnp.dot(p_c, v_c, preferred_element_type=F32))

        s_kr.wait_recv()
        f_kr = remote((FROM_L, 0), (DIAG, 0), 4, right)
        f_kr.start()
        score_chunk(FROM_L, 0)
        s_vl.wait_recv()
        f_vl = remote((FROM_R, 1), (DIAG, 1), 5, left)
        f_vl.start()
        s_kl.wait_recv()
        score_chunk(FROM_R, 1)
        s_vr.wait_recv()
        pv_chunk(FROM_L, 0)
        pv_chunk(FROM_R, 1)

        f_kr.wait_recv()
        f_vl.wait_recv()
        wo_cp.wait()
        for b in range(B):
            r0 = b * S_LOC
            acc = None
            for h in range(H):
                c0 = h * DH
                pv, ps = flash_tile(DIAG, b, h)
                num = out_ref[b, :, c0:c0 + DH] + pv
                den = den_ref[r0:r0 + S_LOC, h:h + 1] + ps
                ctx = (num / den).astype(BF16)
                contrib = jnp.dot(ctx, w_ref[c0:c0 + DH, :].astype(BF16),
                                  preferred_element_type=F32)
                acc = contrib if acc is None else acc + contrib
            out_ref[b] = acc

        for r in (s_kr, s_kl, s_vr, s_vl, f_kr, f_vl):
            r.wait_send()

    return pl.pallas_call(
        body,
        out_shape=jax.ShapeDtypeStruct((B, S_LOC, D), F32),
        in_specs=[
            pl.BlockSpec(memory_space=pltpu.VMEM),
            pl.BlockSpec(memory_space=pl.ANY),
            pl.BlockSpec(memory_space=pl.ANY),
            pl.BlockSpec(memory_space=pl.ANY),
            pl.BlockSpec(memory_space=pl.ANY),
            pl.BlockSpec(memory_space=pltpu.VMEM),
            pl.BlockSpec(memory_space=pltpu.VMEM),
            pl.BlockSpec(memory_space=pltpu.VMEM),
        ],
        out_specs=pl.BlockSpec(memory_space=pltpu.VMEM),
        scratch_shapes=[
            pltpu.VMEM((N_DEV, 2, ROWS, D), BF16),
            pltpu.VMEM((ROWS, D), BF16),
            pltpu.VMEM((D, D), F32),
            pltpu.VMEM((ROWS, DH), F32),
            pltpu.VMEM((2, ROWS, H * S_LOC), BF16),
            pltpu.SemaphoreType.DMA((6,)),
            pltpu.SemaphoreType.DMA((6,)),
            pltpu.SemaphoreType.DMA,
        ],
        compiler_params=pltpu.CompilerParams(
            collective_id=0,
            vmem_limit_bytes=int(63.5 * 1024 * 1024),
        ),
    )(x, Wq, Wk, Wv, Wo, cos, sin, P)


# device time: 101352 ns/iter; 1.1156x vs baseline; 1.1156x over previous
import numpy as np
import jax
import jax.numpy as jnp
from jax import lax
from jax.experimental import pallas as pl
from jax.experimental.pallas import tpu as pltpu

N_DEV = 4
B = 2
S_LOC = 512
D = 1024
H = 8
DH = 128
ROWS = B * S_LOC
SCALE = 0.08838834764831843

F32 = jnp.float32
BF16 = jnp.bfloat16
FP8 = jnp.float8_e4m3fn

OWN, FROM_L, FROM_R = 0, 1, 2


def _rot_mat() -> np.ndarray:
    P = np.zeros((DH, DH), np.float32)
    for i in range(DH // 2):
        P[2 * i + 1, 2 * i] = -1.0
        P[2 * i, 2 * i + 1] = 1.0
    return P


def kernel(x, Wq, Wk, Wv, Wo):
    my = lax.axis_index("i")

    inv = (1.0 / (10000.0 ** (np.arange(0, DH, 2) / DH))).astype(np.float32)
    pos = (my * S_LOC + jnp.arange(S_LOC)).astype(F32)
    ang = pos[:, None] * inv[None, :]
    cos = jnp.repeat(jnp.cos(ang), 2, axis=-1)
    sin = jnp.repeat(jnp.sin(ang), 2, axis=-1)
    P = jnp.asarray(_rot_mat())

    def body(x_ref, wq_hbm, wk_hbm, wv_hbm, wo_hbm, cos_ref, sin_ref, p_ref,
             out_ref, comm_ref, fwd_ref, diag_ref, q_ref, w_ref, den_ref,
             p_buf, send_sems, recv_sems, dma_sem):
        my_pos = lax.axis_index("i")
        right = (my_pos + 1) % N_DEV
        left = (my_pos - 1) % N_DEV

        barrier = pltpu.get_barrier_semaphore()
        for nbr in (left, right):
            pl.semaphore_signal(barrier, inc=1, device_id=(nbr,),
                                device_id_type=pl.DeviceIdType.MESH)

        def load_w(src_hbm):
            cp = pltpu.make_async_copy(src_hbm, w_ref, dma_sem)
            cp.start()
            cp.wait()

        x2 = x_ref[...].reshape(ROWS, D).astype(BF16)
        cs = jnp.concatenate([cos_ref[...], cos_ref[...]], axis=0)
        sn = jnp.concatenate([sin_ref[...], sin_ref[...]], axis=0)

        def proj():
            return jnp.dot(x2, w_ref[...].astype(BF16),
                           preferred_element_type=F32)

        def rope_store(t, store):
            for h in range(H):
                c0 = h * DH
                th = t[:, c0:c0 + DH]
                tr = jnp.dot(th, p_ref[...], preferred_element_type=F32)
                store(c0, (th * cs + tr * sn).astype(BF16))

        def remote(src_ref, dst_ref, sem, dev):
            return pltpu.make_async_remote_copy(
                src_ref=src_ref, dst_ref=dst_ref,
                send_sem=send_sems.at[sem], recv_sem=recv_sems.at[sem],
                device_id=(dev,), device_id_type=pl.DeviceIdType.MESH)

        load_w(wk_hbm)
        rope_store(proj(), lambda c0, t: comm_ref.__setitem__(
            (OWN, 0, slice(None), slice(c0, c0 + DH)), t))
        s_kr = remote(comm_ref.at[OWN, 0], comm_ref.at[FROM_L, 0], 0, right)
        s_kl = remote(comm_ref.at[OWN, 0], comm_ref.at[FROM_R, 0], 1, left)
        pl.semaphore_wait(barrier, 2)
        s_kr.start()
        s_kl.start()

        load_w(wv_hbm)
        comm_ref[OWN, 1] = proj().astype(BF16)
        s_vr = remote(comm_ref.at[OWN, 1], comm_ref.at[FROM_L, 1], 2, right)
        s_vl = remote(comm_ref.at[OWN, 1], comm_ref.at[FROM_R, 1], 3, left)
        s_vr.start()
        s_vl.start()

        load_w(wq_hbm)
        rope_store(proj(), lambda c0, t: q_ref.__setitem__(
            (slice(None), slice(c0, c0 + DH)), t))
        wo_cp = pltpu.make_async_copy(wo_hbm, w_ref, dma_sem)
        wo_cp.start()

        def score_tile(k_c, b, h):
            r0 = b * S_LOC
            c0 = h * DH
            qbh = q_ref[r0:r0 + S_LOC, c0:c0 + DH]
            s_c = lax.dot_general(
                qbh, k_c, (((1,), (1,)), ((), ())),
                preferred_element_type=F32) * SCALE
            return jnp.exp(s_c)

        def flash_chunk(slot, first):
            for b in range(B):
                r0 = b * S_LOC
                for h in range(H):
                    c0 = h * DH
                    p_c = score_tile(
                        comm_ref[slot, 0, r0:r0 + S_LOC, c0:c0 + DH], b, h)
                    pv = jnp.dot(
                        p_c.astype(BF16),
                        comm_ref[slot, 1, r0:r0 + S_LOC, c0:c0 + DH],
                        preferred_element_type=F32)
                    ps = jnp.sum(p_c, axis=-1, keepdims=True)
                    if first:
                        out_ref[b, :, c0:c0 + DH] = pv
                        den_ref[r0:r0 + S_LOC, h:h + 1] = ps
                    else:
                        out_ref[b, :, c0:c0 + DH] = (
                            out_ref[b, :, c0:c0 + DH] + pv)
                        den_ref[r0:r0 + S_LOC, h:h + 1] = (
                            den_ref[r0:r0 + S_LOC, h:h + 1] + ps)

        def score_chunk(slot, pslot):
            for b in range(B):
                r0 = b * S_LOC
                for h in range(H):
                    p_c = score_tile(
                        comm_ref[slot, 0, r0:r0 + S_LOC, h * DH:(h + 1) * DH],
                        b, h)
                    p_buf[pslot, r0:r0 + S_LOC,
                          h * S_LOC:(h + 1) * S_LOC] = p_c.astype(BF16)
                    den_ref[r0:r0 + S_LOC, h:h + 1] = (
                        den_ref[r0:r0 + S_LOC, h:h + 1]
                        + jnp.sum(p_c, axis=-1, keepdims=True))

        def pv_chunk(slot, pslot):
            for b in range(B):
                r0 = b * S_LOC
                for h in range(H):
                    c0 = h * DH
                    v_c = comm_ref[slot, 1, r0:r0 + S_LOC, c0:c0 + DH]
                    p_c = p_buf[pslot, r0:r0 + S_LOC,
                                h * S_LOC:(h + 1) * S_LOC]
                    out_ref[b, :, c0:c0 + DH] = (
                        out_ref[b, :, c0:c0 + DH]
                        + jnp.dot(p_c, v_c, preferred_element_type=F32))

        flash_chunk(OWN, first=True)

        s_kr.wait_recv()
        fwd_ref[0] = comm_ref[FROM_L, 0].astype(FP8)
        f_kr = remote(fwd_ref.at[0], diag_ref.at[0], 4, right)
        f_kr.start()
        score_chunk(FROM_L, 0)
        s_vl.wait_recv()
        fwd_ref[1] = comm_ref[FROM_R, 1].astype(FP8)
        f_vl = remote(fwd_ref.at[1], diag_ref.at[1], 5, left)
        f_vl.start()
        s_kl.wait_recv()
        score_chunk(FROM_R, 1)
        s_vr.wait_recv()
        pv_chunk(FROM_L, 0)
        pv_chunk(FROM_R, 1)

        f_kr.wait_recv()
        f_vl.wait_recv()
        wo_cp.wait()
        for b in range(B):
            r0 = b * S_LOC
            acc = None
            for h in range(H):
                c0 = h * DH
                p_c = score_tile(
                    diag_ref[0, r0:r0 + S_LOC, c0:c0 + DH].astype(BF16),
                    b, h)
                pv = jnp.dot(
                    p_c.astype(BF16),
                    diag_ref[1, r0:r0 + S_LOC, c0:c0 + DH].astype(BF16),
                    preferred_element_type=F32)
                ps = jnp.sum(p_c, axis=-1, keepdims=True)
                num = out_ref[b, :, c0:c0 + DH] + pv
                den = den_ref[r0:r0 + S_LOC, h:h + 1] + ps
                ctx = (num / den).astype(BF16)
                contrib = jnp.dot(ctx, w_ref[c0:c0 + DH, :].astype(BF16),
                                  preferred_element_type=F32)
                acc = contrib if acc is None else acc + contrib
            out_ref[b] = acc

        for r in (s_kr, s_kl, s_vr, s_vl, f_kr, f_vl):
            r.wait_send()

    return pl.pallas_call(
        body,
        out_shape=jax.ShapeDtypeStruct((B, S_LOC, D), F32),
        in_specs=[
            pl.BlockSpec(memory_space=pltpu.VMEM),
            pl.BlockSpec(memory_space=pl.ANY),
            pl.BlockSpec(memory_space=pl.ANY),
            pl.BlockSpec(memory_space=pl.ANY),
            pl.BlockSpec(memory_space=pl.ANY),
            pl.BlockSpec(memory_space=pltpu.VMEM),
            pl.BlockSpec(memory_space=pltpu.VMEM),
            pl.BlockSpec(memory_space=pltpu.VMEM),
        ],
        out_specs=pl.BlockSpec(memory_space=pltpu.VMEM),
        scratch_shapes=[
            pltpu.VMEM((3, 2, ROWS, D), BF16),
            pltpu.VMEM((2, ROWS, D), FP8),
            pltpu.VMEM((2, ROWS, D), FP8),
            pltpu.VMEM((ROWS, D), BF16),
            pltpu.VMEM((D, D), F32),
            pltpu.VMEM((ROWS, DH), F32),
            pltpu.VMEM((2, ROWS, H * S_LOC), BF16),
            pltpu.SemaphoreType.DMA((6,)),
            pltpu.SemaphoreType.DMA((6,)),
            pltpu.SemaphoreType.DMA,
        ],
        compiler_params=pltpu.CompilerParams(
            collective_id=0,
            vmem_limit_bytes=int(63.5 * 1024 * 1024),
        ),
    )(x, Wq, Wk, Wv, Wo, cos, sin, P)
